# baseline (speedup 1.0000x reference)
import functools

import jax
import jax.numpy as jnp
from jax import lax
from jax.experimental import pallas as pl
from jax.experimental.pallas import tpu as pltpu
from jax.experimental.pallas import tpu_sc as plsc

_N = 50000
_D = 128
_S_A = 8
_S_S = 4
_NC = 2
_NS = 16
_NW = _NC * _NS
_N_PAD = 53248


_MESH = plsc.VectorSubcoreMesh(
    core_axis_name="c", subcore_axis_name="s",
    num_cores=_NC, num_subcores=_NS)
_SC_PARAMS = pltpu.CompilerParams(needs_layout_passes=False)


@functools.lru_cache(maxsize=None)
def _make_sc_alpha(S):
    dpc = 128 // S
    dwork = _N_PAD // _NW
    nch = dwork * S // 128
    nhalf = dpc // 16
    GB = 4
    assert nch % GB == 0

    @functools.partial(
        pl.kernel,
        mesh=_MESH,
        compiler_params=_SC_PARAMS,
        out_type=jax.ShapeDtypeStruct((_N_PAD * S,), jnp.float32),
        scratch_types=[
            pltpu.VMEM((_N_PAD,), jnp.float32),
            pltpu.VMEM((dwork,), jnp.float32),
            pltpu.VMEM((nch * 128,), jnp.int32),
            pltpu.VMEM((GB * 128,), jnp.float32),
        ],
    )
    def alpha_kernel(el1, er1, idxf, out, el_v, er_v, idxf_v, astage):
        wid = lax.axis_index("s") * _NC + lax.axis_index("c")
        pltpu.sync_copy(el1, el_v)
        pltpu.sync_copy(er1.at[pl.ds(wid * dwork, dwork)], er_v)
        pltpu.sync_copy(idxf.at[pl.ds(wid * nch * 128, nch * 128)], idxf_v)
        iota = lax.iota(jnp.int32, 16)

        def body(kb, carry):
            for q in range(GB):
                c = kb * GB + q
                for h in range(nhalf):
                    t = c * dpc + h * 16 + iota
                    erh = plsc.load_gather(er_v, [t])
                    e_js = []
                    for j in range(S):
                        pos = c * 128 + (h * 16) * S + iota * S + j
                        nbr = plsc.load_gather(idxf_v, [pos])
                        elj = plsc.load_gather(el_v, [nbr])
                        e = elj + erh
                        e_js.append(jnp.where(e >= 0.0, e, 0.2 * e))
                    m = functools.reduce(jnp.maximum, e_js)
                    p_js = [jnp.exp(e - m) for e in e_js]
                    rinv = 1.0 / functools.reduce(jnp.add, p_js)
                    for j in range(S):
                        pos = q * 128 + (h * 16) * S + iota * S + j
                        plsc.store_scatter(astage, [pos], p_js[j] * rinv)
            pltpu.sync_copy(
                astage,
                out.at[pl.ds(wid * nch * 128 + kb * GB * 128, GB * 128)])
            return carry

        lax.fori_loop(0, nch // GB, body, 0)

    return alpha_kernel


@functools.lru_cache(maxsize=None)
def _make_sc_wsum(S):
    dpc = 128 // S
    dwork = _N_PAD // _NW
    nch = dwork * S // 128
    NB = 4
    obr = NB * dpc
    assert nch % NB == 0

    @functools.partial(
        pl.kernel,
        mesh=_MESH,
        compiler_params=_SC_PARAMS,
        out_type=jax.ShapeDtypeStruct((_N_PAD, _D), jnp.float32),
        scratch_types=[
            pltpu.VMEM((nch * 128,), jnp.float32),
            pltpu.VMEM((nch, 128), jnp.int32),
            pltpu.VMEM((128,), jnp.float32),
            pltpu.VMEM((obr, _D), jnp.float32),
            [pltpu.VMEM((128, _D), jnp.float32)] * NB,
            [pltpu.SemaphoreType.DMA] * NB,
        ],
    )
    def wsum_kernel(table, alphas, idx3, bias, out,
                    alpha_v, idx_v, b_v, outbuf, stages, sems):
        wid = lax.axis_index("s") * _NC + lax.axis_index("c")
        pltpu.sync_copy(alphas.at[pl.ds(wid * nch * 128, nch * 128)], alpha_v)
        pltpu.sync_copy(idx3.at[wid], idx_v)
        pltpu.sync_copy(bias, b_v)

        def issue(c, q):
            return pltpu.async_copy(table.at[idx_v.at[c]], stages[q], sems[q])

        for q in range(NB):
            issue(q, q)

        def body(kb, carry):
            for q in range(NB):
                c = kb * NB + q
                pltpu.make_async_copy(
                    table.at[pl.ds(0, 128)], stages[q], sems[q]).wait()
                st = stages[q]
                for i in range(dpc):
                    accs = [b_v[pl.ds(d * 16, 16)] for d in range(8)]
                    for j in range(S):
                        a = plsc.load_gather(
                            alpha_v,
                            [jnp.full((16,), c * 128 + i * S + j, jnp.int32)])
                        r = i * S + j
                        for d in range(8):
                            accs[d] = accs[d] + a * st[r, pl.ds(d * 16, 16)]
                    for d in range(8):
                        outbuf[q * dpc + i, pl.ds(d * 16, 16)] = accs[d]

                @pl.when(c + NB < nch)
                def _():
                    issue(c + NB, q)

            pltpu.sync_copy(
                outbuf, out.at[pl.ds(wid * dwork + kb * obr, obr)])
            return carry

        lax.fori_loop(0, nch // NB, body, 0)

    return wsum_kernel


def _proj_kernel(ha_ref, hs_ref, hp_ref,
                 wa_ref, ws_ref, wda_ref, wds_ref,
                 ala_ref, als_ref, ara_ref, ars_ref,
                 fa_ref, fs_ref, ela_ref, els_ref, era_ref, ers_ref):
    dn = (((1,), (1,)), ((), ()))
    fa = lax.dot_general(ha_ref[...], wa_ref[...], dn,
                         preferred_element_type=jnp.float32)
    fs = lax.dot_general(hs_ref[...], ws_ref[...], dn,
                         preferred_element_type=jnp.float32)
    fa_ref[...] = fa
    fs_ref[...] = fs
    ela_ref[...] = jnp.sum(
        fa.reshape(8, 128, _D) * ala_ref[...].reshape(1, 1, _D), axis=-1)
    els_ref[...] = jnp.sum(
        fs.reshape(8, 128, _D) * als_ref[...].reshape(1, 1, _D), axis=-1)
    dn2 = (((1,), (0,)), ((), ()))
    wva = lax.dot_general(ara_ref[...], wda_ref[...], dn2,
                          preferred_element_type=jnp.float32)
    wvs = lax.dot_general(ars_ref[...], wds_ref[...], dn2,
                          preferred_element_type=jnp.float32)
    hp3 = hp_ref[...].reshape(8, 128, _D)
    era_ref[...] = jnp.sum(hp3 * wva.reshape(1, 1, _D), axis=-1)
    ers_ref[...] = jnp.sum(hp3 * wvs.reshape(1, 1, _D), axis=-1)


def _beta_kernel(e0_ref, e1_ref, wfc_ref, bfc_ref, att_ref, l0_ref, l1_ref):
    @pl.when(pl.program_id(0) == 0)
    def _():
        l0_ref[0, 0] = 0.0
        l1_ref[0, 0] = 0.0

    dn = (((1,), (1,)), ((), ()))

    def part(e):
        t = jnp.tanh(
            lax.dot_general(e, wfc_ref[...], dn,
                            preferred_element_type=jnp.float32)
            + bfc_ref[...])
        return jnp.sum(t * att_ref[...])

    l0_ref[0, 0] += part(e0_ref[...])
    l1_ref[0, 0] += part(e1_ref[...])


def _combine_kernel(l0_ref, l1_ref, e0_ref, e1_ref, z_ref):
    d = (l1_ref[0, 0] - l0_ref[0, 0]) * (1.0 / _N)
    e0 = e0_ref[...]
    beta0 = 1.0 / (1.0 + jnp.exp(jnp.full(e0.shape, d, jnp.float32)))
    z_ref[...] = beta0 * e0 + (1.0 - beta0) * e1_ref[...]


def _flat_idx(nbr, S):
    nbr = nbr.astype(jnp.int32)
    pad = jnp.zeros((_N_PAD - _N, S), jnp.int32)
    return jnp.concatenate([nbr, pad], axis=0).reshape(-1)


def kernel(h_paper, h_author, h_subject,
           Wsrc_a, Wdst_a, al_a, ar_a, b_a,
           Wsrc_s, Wdst_s, al_s, ar_s, b_s,
           W_fc, b_fc, att,
           nbr_author, nbr_subject):
    pad = ((0, _N_PAD - _N), (0, 0))
    ha = jnp.pad(h_author, pad)
    hs = jnp.pad(h_subject, pad)
    hp = jnp.pad(h_paper, pad)

    BN1 = 1024
    fullmat = pl.BlockSpec((_D, _D), lambda i: (0, 0))
    vec2 = pl.BlockSpec((1, _D), lambda i: (0, 0))
    hblk = pl.BlockSpec((BN1, _D), lambda i: (i, 0))
    eblk = pl.BlockSpec((8, 128), lambda i: (i, 0))
    erows = _N_PAD // 128
    fs_a, fs_s, el_a, el_s, er_a, er_s = pl.pallas_call(
        _proj_kernel,
        grid=(_N_PAD // BN1,),
        in_specs=[hblk, hblk, hblk, fullmat, fullmat, fullmat, fullmat,
                  vec2, vec2, vec2, vec2],
        out_specs=[hblk, hblk, eblk, eblk, eblk, eblk],
        out_shape=[jax.ShapeDtypeStruct((_N_PAD, _D), jnp.float32)] * 2
        + [jax.ShapeDtypeStruct((erows, 128), jnp.float32)] * 4,
    )(ha, hs, hp, Wsrc_a, Wsrc_s, Wdst_a, Wdst_s,
      al_a.reshape(1, _D), al_s.reshape(1, _D),
      ar_a.reshape(1, _D), ar_s.reshape(1, _D))

    idx_a = _flat_idx(nbr_author, _S_A)
    idx_s = _flat_idx(nbr_subject, _S_S)
    al_a_e = _make_sc_alpha(_S_A)(el_a.reshape(-1), er_a.reshape(-1), idx_a)
    al_s_e = _make_sc_alpha(_S_S)(el_s.reshape(-1), er_s.reshape(-1), idx_s)
    e0 = _make_sc_wsum(_S_A)(fs_a, al_a_e, idx_a.reshape(_NW, -1, 128), b_a)
    e1 = _make_sc_wsum(_S_S)(fs_s, al_s_e, idx_s.reshape(_NW, -1, 128), b_s)

    BN = 400
    grid = (_N // BN,)
    nblk = pl.BlockSpec((BN, _D), lambda i: (i, 0))

    l0, l1 = pl.pallas_call(
        _beta_kernel,
        grid=grid,
        in_specs=[nblk, nblk, fullmat, vec2, vec2],
        out_specs=[pl.BlockSpec(memory_space=pltpu.SMEM)] * 2,
        out_shape=[jax.ShapeDtypeStruct((1, 1), jnp.float32)] * 2,
    )(e0, e1, W_fc, b_fc.reshape(1, _D), att.reshape(1, _D))

    z = pl.pallas_call(
        _combine_kernel,
        grid=grid,
        in_specs=[pl.BlockSpec(memory_space=pltpu.SMEM)] * 2 + [nblk, nblk],
        out_specs=nblk,
        out_shape=jax.ShapeDtypeStruct((_N, _D), jnp.float32),
    )(l0, l1, e0, e1)

    return z


# speedup vs baseline: 1.8282x; 1.8282x over previous
import functools

import jax
import jax.numpy as jnp
from jax import lax
from jax.experimental import pallas as pl
from jax.experimental.pallas import tpu as pltpu
from jax.experimental.pallas import tpu_sc as plsc

_N = 50000
_D = 128
_S_A = 8
_S_S = 4
_N_PAD = 50176
_NC = 2
_NS = 16
_NW = _NC * _NS


@functools.lru_cache(maxsize=None)
def _make_sc_gather(S, K):
    B = _N_PAD * S
    bpw = B // _NW
    nch = bpw // 128
    nbody = nch // K
    assert nch % K == 0

    mesh = plsc.VectorSubcoreMesh(
        core_axis_name="c", subcore_axis_name="s",
        num_cores=_NC, num_subcores=_NS)

    @functools.partial(
        pl.kernel,
        mesh=mesh,
        out_type=jax.ShapeDtypeStruct((B, _D), jnp.float32),
        scratch_types=[
            pltpu.VMEM((nch, 128), jnp.int32),
            pltpu.VMEM((K * 128, _D), jnp.float32),
            pltpu.SemaphoreType.DMA,
        ],
    )
    def gather(table, idx, out, idx_v, stage, sem):
        wid = lax.axis_index("s") * _NC + lax.axis_index("c")
        pltpu.sync_copy(idx.at[wid], idx_v)

        def body(i, carry):
            cps = [
                pltpu.async_copy(
                    table.at[idx_v.at[i * K + b]],
                    stage.at[pl.ds(b * 128, 128)],
                    sem,
                )
                for b in range(K)
            ]
            for c in cps:
                c.wait()
            pltpu.sync_copy(
                stage, out.at[pl.ds(wid * bpw + i * (K * 128), K * 128)]
            )
            return carry

        lax.fori_loop(0, nbody, body, 0)

    return gather


def _proj_kernel(ha_ref, hs_ref, wa_ref, ws_ref, fa_ref, fs_ref):
    dn = (((1,), (1,)), ((), ()))
    fa_ref[...] = lax.dot_general(
        ha_ref[...], wa_ref[...], dn, preferred_element_type=jnp.float32)
    fs_ref[...] = lax.dot_general(
        hs_ref[...], ws_ref[...], dn, preferred_element_type=jnp.float32)


def _gat_tail_kernel(fsg_ref, hp_ref, wdst_ref, al_ref, ar_ref, b_ref,
                     wfc_ref, bfc_ref, att_ref, out_ref, l_ref):
    fsg = fsg_ref[...]
    el_g = jnp.sum(fsg * al_ref[...], axis=-1)
    wv = lax.dot_general(
        ar_ref[...], wdst_ref[...], (((1,), (0,)), ((), ())),
        preferred_element_type=jnp.float32)
    er = jnp.sum(hp_ref[...] * wv, axis=-1, keepdims=True)
    e = el_g + er
    e = jnp.where(e >= 0.0, e, 0.2 * e)
    m = jnp.max(e, axis=-1, keepdims=True)
    p = jnp.exp(e - m)
    alpha = p / jnp.sum(p, axis=-1, keepdims=True)
    out = jnp.sum(alpha[:, :, None] * fsg, axis=1) + b_ref[...]
    out_ref[...] = out

    @pl.when(pl.program_id(0) == 0)
    def _():
        l_ref[0, 0] = 0.0

    t = jnp.tanh(
        lax.dot_general(out, wfc_ref[...], (((1,), (1,)), ((), ())),
                        preferred_element_type=jnp.float32)
        + bfc_ref[...])
    l_ref[0, 0] += jnp.sum(t * att_ref[...])


def _combine_kernel(l0_ref, l1_ref, e0_ref, e1_ref, z_ref):
    d = (l1_ref[0, 0] - l0_ref[0, 0]) * (1.0 / _N)
    e0 = e0_ref[...]
    beta0 = 1.0 / (1.0 + jnp.exp(jnp.full(e0.shape, d, jnp.float32)))
    z_ref[...] = beta0 * e0 + (1.0 - beta0) * e1_ref[...]


def _flat_idx(nbr, S):
    nbr = nbr.astype(jnp.int32)
    pad = jnp.zeros((_N_PAD - _N, S), jnp.int32)
    return jnp.concatenate([nbr, pad], axis=0).reshape(_NW, -1, 128)


def kernel(h_paper, h_author, h_subject,
           Wsrc_a, Wdst_a, al_a, ar_a, b_a,
           Wsrc_s, Wdst_s, al_s, ar_s, b_s,
           W_fc, b_fc, att,
           nbr_author, nbr_subject):
    BN1 = 1000
    fs_a, fs_s = pl.pallas_call(
        _proj_kernel,
        grid=(_N // BN1,),
        in_specs=[
            pl.BlockSpec((BN1, _D), lambda i: (i, 0)),
            pl.BlockSpec((BN1, _D), lambda i: (i, 0)),
            pl.BlockSpec((_D, _D), lambda i: (0, 0)),
            pl.BlockSpec((_D, _D), lambda i: (0, 0)),
        ],
        out_specs=[pl.BlockSpec((BN1, _D), lambda i: (i, 0))] * 2,
        out_shape=[jax.ShapeDtypeStruct((_N, _D), jnp.float32)] * 2,
    )(h_author, h_subject, Wsrc_a, Wsrc_s)

    fsg_a = _make_sc_gather(_S_A, 7)(fs_a, _flat_idx(nbr_author, _S_A))
    fsg_s = _make_sc_gather(_S_S, 7)(fs_s, _flat_idx(nbr_subject, _S_S))

    BN = 400
    grid = (_N // BN,)

    def gat_tail(fsg, S, Wdst, al, ar, b):
        return pl.pallas_call(
            _gat_tail_kernel,
            grid=grid,
            in_specs=[
                pl.BlockSpec((BN, S, _D), lambda i: (i, 0, 0)),
                pl.BlockSpec((BN, _D), lambda i: (i, 0)),
                pl.BlockSpec((_D, _D), lambda i: (0, 0)),
                pl.BlockSpec((1, 1, _D), lambda i: (0, 0, 0)),
                pl.BlockSpec((1, _D), lambda i: (0, 0)),
                pl.BlockSpec((1, _D), lambda i: (0, 0)),
                pl.BlockSpec((_D, _D), lambda i: (0, 0)),
                pl.BlockSpec((1, _D), lambda i: (0, 0)),
                pl.BlockSpec((1, _D), lambda i: (0, 0)),
            ],
            out_specs=[pl.BlockSpec((BN, _D), lambda i: (i, 0)),
                       pl.BlockSpec(memory_space=pltpu.SMEM)],
            out_shape=[jax.ShapeDtypeStruct((_N, _D), jnp.float32),
                       jax.ShapeDtypeStruct((1, 1), jnp.float32)],
        )(fsg.reshape(_N_PAD, S, _D), h_paper, Wdst,
          al.reshape(1, 1, _D), ar.reshape(1, _D), b.reshape(1, _D),
          W_fc, b_fc.reshape(1, _D), att.reshape(1, _D))

    e0, l0 = gat_tail(fsg_a, _S_A, Wdst_a, al_a, ar_a, b_a)
    e1, l1 = gat_tail(fsg_s, _S_S, Wdst_s, al_s, ar_s, b_s)

    z = pl.pallas_call(
        _combine_kernel,
        grid=grid,
        in_specs=[
            pl.BlockSpec(memory_space=pltpu.SMEM),
            pl.BlockSpec(memory_space=pltpu.SMEM),
            pl.BlockSpec((BN, _D), lambda i: (i, 0)),
            pl.BlockSpec((BN, _D), lambda i: (i, 0)),
        ],
        out_specs=pl.BlockSpec((BN, _D), lambda i: (i, 0)),
        out_shape=jax.ShapeDtypeStruct((_N, _D), jnp.float32),
    )(l0, l1, e0, e1)

    return z


# speedup vs baseline: 1.8814x; 1.0291x over previous
import functools

import jax
import jax.numpy as jnp
from jax import lax
from jax.experimental import pallas as pl
from jax.experimental.pallas import tpu as pltpu
from jax.experimental.pallas import tpu_sc as plsc

_N = 50000
_D = 128
_S_A = 8
_S_S = 4
_N_PAD = 50176
_NC = 2
_NS = 16
_NW = _NC * _NS


@functools.lru_cache(maxsize=None)
def _make_sc_gather(S, K):
    B = _N_PAD * S
    bpw = B // _NW
    nch = bpw // 128
    nbody = nch // K
    assert nch % K == 0

    mesh = plsc.VectorSubcoreMesh(
        core_axis_name="c", subcore_axis_name="s",
        num_cores=_NC, num_subcores=_NS)

    @functools.partial(
        pl.kernel,
        mesh=mesh,
        out_type=jax.ShapeDtypeStruct((B, _D), jnp.float32),
        scratch_types=[
            pltpu.VMEM((nch, 128), jnp.int32),
            pltpu.VMEM((K * 128, _D), jnp.float32),
            pltpu.SemaphoreType.DMA,
        ],
    )
    def gather(table, idx, out, idx_v, stage, sem):
        wid = lax.axis_index("s") * _NC + lax.axis_index("c")
        pltpu.sync_copy(idx.at[wid], idx_v)

        def body(i, carry):
            cps = [
                pltpu.async_copy(
                    table.at[idx_v.at[i * K + b]],
                    stage.at[pl.ds(b * 128, 128)],
                    sem,
                )
                for b in range(K)
            ]
            for c in cps:
                c.wait()
            pltpu.sync_copy(
                stage, out.at[pl.ds(wid * bpw + i * (K * 128), K * 128)]
            )
            return carry

        lax.fori_loop(0, nbody, body, 0)

    return gather


def _proj_kernel(ha_ref, hs_ref, wa_ref, ws_ref, fa_ref, fs_ref):
    dn = (((1,), (1,)), ((), ()))
    fa_ref[...] = lax.dot_general(
        ha_ref[...], wa_ref[...], dn, preferred_element_type=jnp.float32)
    fs_ref[...] = lax.dot_general(
        hs_ref[...], ws_ref[...], dn, preferred_element_type=jnp.float32)


def _gat_tail_kernel(fsga_ref, fsgs_ref, hp_ref, wda_ref, wds_ref,
                     ala_ref, als_ref, ara_ref, ars_ref, ba_ref, bs_ref,
                     wfc_ref, bfc_ref, att_ref,
                     e0_ref, e1_ref, l0_ref, l1_ref):
    hp = hp_ref[...]

    def one(fsg, wd_ref, al_ref, ar_ref, b_ref, out_ref, l_ref):
        el_g = jnp.sum(fsg * al_ref[...], axis=-1)
        wv = lax.dot_general(
            ar_ref[...], wd_ref[...], (((1,), (0,)), ((), ())),
            preferred_element_type=jnp.float32)
        er = jnp.sum(hp * wv, axis=-1, keepdims=True)
        e = el_g + er
        e = jnp.where(e >= 0.0, e, 0.2 * e)
        m = jnp.max(e, axis=-1, keepdims=True)
        p = jnp.exp(e - m)
        alpha = p / jnp.sum(p, axis=-1, keepdims=True)
        out = jnp.sum(alpha[:, :, None] * fsg, axis=1) + b_ref[...]
        out_ref[...] = out

        @pl.when(pl.program_id(0) == 0)
        def _():
            l_ref[0, 0] = 0.0

        t = jnp.tanh(
            lax.dot_general(out, wfc_ref[...], (((1,), (1,)), ((), ())),
                            preferred_element_type=jnp.float32)
            + bfc_ref[...])
        l_ref[0, 0] += jnp.sum(t * att_ref[...])

    one(fsga_ref[...], wda_ref, ala_ref, ara_ref, ba_ref, e0_ref, l0_ref)
    one(fsgs_ref[...], wds_ref, als_ref, ars_ref, bs_ref, e1_ref, l1_ref)


def _combine_kernel(l0_ref, l1_ref, e0_ref, e1_ref, z_ref):
    d = (l1_ref[0, 0] - l0_ref[0, 0]) * (1.0 / _N)
    e0 = e0_ref[...]
    beta0 = 1.0 / (1.0 + jnp.exp(jnp.full(e0.shape, d, jnp.float32)))
    z_ref[...] = beta0 * e0 + (1.0 - beta0) * e1_ref[...]


def _flat_idx(nbr, S):
    nbr = nbr.astype(jnp.int32)
    pad = jnp.zeros((_N_PAD - _N, S), jnp.int32)
    return jnp.concatenate([nbr, pad], axis=0).reshape(_NW, -1, 128)


def kernel(h_paper, h_author, h_subject,
           Wsrc_a, Wdst_a, al_a, ar_a, b_a,
           Wsrc_s, Wdst_s, al_s, ar_s, b_s,
           W_fc, b_fc, att,
           nbr_author, nbr_subject):
    BN1 = 1000
    fs_a, fs_s = pl.pallas_call(
        _proj_kernel,
        grid=(_N // BN1,),
        in_specs=[
            pl.BlockSpec((BN1, _D), lambda i: (i, 0)),
            pl.BlockSpec((BN1, _D), lambda i: (i, 0)),
            pl.BlockSpec((_D, _D), lambda i: (0, 0)),
            pl.BlockSpec((_D, _D), lambda i: (0, 0)),
        ],
        out_specs=[pl.BlockSpec((BN1, _D), lambda i: (i, 0))] * 2,
        out_shape=[jax.ShapeDtypeStruct((_N, _D), jnp.float32)] * 2,
    )(h_author, h_subject, Wsrc_a, Wsrc_s)

    fsg_a = _make_sc_gather(_S_A, 7)(fs_a, _flat_idx(nbr_author, _S_A))
    fsg_s = _make_sc_gather(_S_S, 7)(fs_s, _flat_idx(nbr_subject, _S_S))

    BN = 1000
    grid = (_N // BN,)
    mat = pl.BlockSpec((_D, _D), lambda i: (0, 0))
    vec3 = pl.BlockSpec((1, 1, _D), lambda i: (0, 0, 0))
    vec2 = pl.BlockSpec((1, _D), lambda i: (0, 0))
    nblk = pl.BlockSpec((BN, _D), lambda i: (i, 0))
    smem = pl.BlockSpec(memory_space=pltpu.SMEM)

    e0, e1, l0, l1 = pl.pallas_call(
        _gat_tail_kernel,
        grid=grid,
        in_specs=[
            pl.BlockSpec((BN, _S_A, _D), lambda i: (i, 0, 0)),
            pl.BlockSpec((BN, _S_S, _D), lambda i: (i, 0, 0)),
            nblk, mat, mat, vec3, vec3, vec2, vec2, vec2, vec2,
            mat, vec2, vec2,
        ],
        out_specs=[nblk, nblk, smem, smem],
        out_shape=[jax.ShapeDtypeStruct((_N, _D), jnp.float32)] * 2
        + [jax.ShapeDtypeStruct((1, 1), jnp.float32)] * 2,
    )(fsg_a.reshape(_N_PAD, _S_A, _D), fsg_s.reshape(_N_PAD, _S_S, _D),
      h_paper, Wdst_a, Wdst_s,
      al_a.reshape(1, 1, _D), al_s.reshape(1, 1, _D),
      ar_a.reshape(1, _D), ar_s.reshape(1, _D),
      b_a.reshape(1, _D), b_s.reshape(1, _D),
      W_fc, b_fc.reshape(1, _D), att.reshape(1, _D))

    z = pl.pallas_call(
        _combine_kernel,
        grid=grid,
        in_specs=[
            pl.BlockSpec(memory_space=pltpu.SMEM),
            pl.BlockSpec(memory_space=pltpu.SMEM),
            pl.BlockSpec((BN, _D), lambda i: (i, 0)),
            pl.BlockSpec((BN, _D), lambda i: (i, 0)),
        ],
        out_specs=pl.BlockSpec((BN, _D), lambda i: (i, 0)),
        out_shape=jax.ShapeDtypeStruct((_N, _D), jnp.float32),
    )(l0, l1, e0, e1)

    return z
